# baseline (speedup 1.0000x reference)
import math

import jax
import jax.numpy as jnp
from jax import lax
from jax.experimental import pallas as pl
from jax.experimental.pallas import tpu as pltpu
from jax.experimental.pallas import tpu_sc as plsc

EMB = 32
SCALE = math.sqrt(EMB)


CBLK = 4000


def _tc_compact(emb):
    V = emb.shape[0]
    n = V // CBLK

    def body(x_ref, o_ref):
        for q in range(4):
            o_ref[:, q * EMB:(q + 1) * EMB] = x_ref[q::4, :] * SCALE

    return pl.pallas_call(
        body,
        grid=(n,),
        in_specs=[pl.BlockSpec((CBLK, EMB), lambda i: (i, 0))],
        out_specs=pl.BlockSpec((CBLK * EMB // 128, 128), lambda i: (i, 0)),
        out_shape=jax.ShapeDtypeStruct((V * EMB // 128, 128), jnp.float32),
    )(emb)


FBLK = 1000


def _tc_format(out128, B):
    n = out128.shape[0] // FBLK

    def body(x_ref, o_ref):
        for q in range(4):
            o_ref[q::4, :] = x_ref[:, q * EMB:(q + 1) * EMB]

    return pl.pallas_call(
        body,
        grid=(n,),
        in_specs=[pl.BlockSpec((FBLK, 128), lambda i: (i, 0))],
        out_specs=pl.BlockSpec((FBLK * 4, EMB), lambda i: (i, 0)),
        out_shape=jax.ShapeDtypeStruct((B, EMB), jnp.float32),
    )(out128)


K = 128
CH = 5
C = K * CH


def _make_gather(B, V):
    info = plsc.get_sparse_core_info()
    NC = info.num_cores
    NW = NC * info.num_subcores
    RW = B // NW
    NCH = RW // C
    assert NCH % 2 == 0 and NCH * C == RW

    mesh = plsc.VectorSubcoreMesh(core_axis_name="c", subcore_axis_name="s")

    @pl.kernel(
        mesh=mesh,
        out_type=jax.ShapeDtypeStruct((B * EMB // 128, 128), jnp.float32),
        scratch_types=[
            pltpu.VMEM((RW // K, K), jnp.int32),
            pltpu.VMEM((2, C, EMB), jnp.float32),
            pltpu.VMEM((2, C * EMB // 128, 128), jnp.float32),
            pltpu.SemaphoreType.DMA,
            pltpu.SemaphoreType.DMA,
            pltpu.SemaphoreType.DMA,
            pltpu.SemaphoreType.DMA,
        ],
        compiler_params=pltpu.CompilerParams(use_tc_tiling_on_sc=False),
    )
    def k(tok_hbm, tbl_hbm, out_hbm, idx_v, rows_v, r128_v, sg0, sg1, so0, so1):
        wid = lax.axis_index("s") * NC + lax.axis_index("c")
        base = wid * RW
        semg = (sg0, sg1)
        semo = (so0, so1)

        pltpu.sync_copy(
            tok_hbm.at[pl.ds(pl.multiple_of(base // K, 8), RW // K)], idx_v)

        def fire(ci, b):
            for j in range(CH):
                pltpu.async_copy(
                    tbl_hbm.at[idx_v.at[ci * CH + j]],
                    rows_v.at[b, pl.ds(j * K, K)],
                    semg[b],
                )

        def wait_g(b):
            pltpu.make_async_copy(
                tbl_hbm.at[pl.ds(0, C)], rows_v.at[b], semg[b]).wait()

        def out_slice(ci):
            r0 = pl.multiple_of((base + ci * C) * EMB // 128, 8)
            return out_hbm.at[pl.ds(r0, C * EMB // 128)]

        def repack(b):
            def body(i, carry):
                r = i * 16
                vals = []
                for u in range(16):
                    for h in range(EMB // 16):
                        vals.append(rows_v[b, r + u, pl.ds(h * 16, 16)])
                for u in range(16):
                    for h in range(EMB // 16):
                        d = (b, r // 4 + u // 4,
                             pl.ds((u % 4) * EMB + h * 16, 16))
                        r128_v[d] = vals[u * (EMB // 16) + h]
                return carry
            lax.fori_loop(0, C // 16, body, 0)

        fire(0, 0)

        def outer(c2, carry):
            for b in range(2):
                ci = c2 * 2 + b
                nb = 1 - b

                @pl.when(ci + 1 < NCH)
                def _fire_next():
                    fire(ci + 1, nb)

                wait_g(b)

                @pl.when(ci >= 2)
                def _drain_out():
                    pltpu.make_async_copy(
                        r128_v.at[b], out_slice(ci - 2), semo[b]).wait()

                repack(b)
                pltpu.async_copy(r128_v.at[b], out_slice(ci), semo[b])
            return carry

        lax.fori_loop(0, NCH // 2, outer, 0)
        pltpu.make_async_copy(r128_v.at[0], out_slice(NCH - 2), semo[0]).wait()
        pltpu.make_async_copy(r128_v.at[1], out_slice(NCH - 1), semo[1]).wait()

    return k


def kernel(tokens, embedding):
    B = tokens.shape[0] * tokens.shape[1]
    V = embedding.shape[0]
    tok2d = tokens.reshape(B // K, K).astype(jnp.int32)
    tbl = _tc_compact(embedding)
    out128 = _make_gather(B, V)(tok2d, tbl.reshape(V, EMB))
    return out128.reshape(tokens.shape[0], tokens.shape[1], EMB)


# speedup vs baseline: 1.1806x; 1.1806x over previous
import math

import jax
import jax.numpy as jnp
from jax import lax
from jax.experimental import pallas as pl
from jax.experimental.pallas import tpu as pltpu
from jax.experimental.pallas import tpu_sc as plsc

EMB = 32
SCALE = math.sqrt(EMB)


K = 128
CH = 5
C = K * CH


def _make_gather(B, V):
    info = plsc.get_sparse_core_info()
    NC = info.num_cores
    NW = NC * info.num_subcores
    RW = B // NW
    NCH = RW // C
    assert NCH % 2 == 0 and NCH * C == RW

    mesh = plsc.VectorSubcoreMesh(core_axis_name="c", subcore_axis_name="s")

    @pl.kernel(
        mesh=mesh,
        out_type=jax.ShapeDtypeStruct((B * EMB // 128, 128), jnp.float32),
        scratch_types=[
            pltpu.VMEM((RW // K, K), jnp.int32),
            pltpu.VMEM((2, C, EMB), jnp.float32),
            pltpu.VMEM((2, C * EMB // 128, 128), jnp.float32),
            pltpu.SemaphoreType.DMA,
            pltpu.SemaphoreType.DMA,
            pltpu.SemaphoreType.DMA,
            pltpu.SemaphoreType.DMA,
        ],
        compiler_params=pltpu.CompilerParams(use_tc_tiling_on_sc=False),
    )
    def k(tok_hbm, tbl_hbm, out_hbm, idx_v, rows_v, r128_v, sg0, sg1, so0, so1):
        wid = lax.axis_index("s") * NC + lax.axis_index("c")
        base = wid * RW
        semg = (sg0, sg1)
        semo = (so0, so1)

        pltpu.sync_copy(
            tok_hbm.at[pl.ds(pl.multiple_of(base // K, 8), RW // K)], idx_v)

        def fire(ci, b):
            for j in range(CH):
                pltpu.async_copy(
                    tbl_hbm.at[idx_v.at[ci * CH + j]],
                    rows_v.at[b, pl.ds(j * K, K)],
                    semg[b],
                )

        def wait_g(b):
            pltpu.make_async_copy(
                tbl_hbm.at[pl.ds(0, C)], rows_v.at[b], semg[b]).wait()

        def out_slice(ci):
            r0 = pl.multiple_of((base + ci * C) * EMB // 128, 8)
            return out_hbm.at[pl.ds(r0, C * EMB // 128)]

        def repack(b):
            def body(i, carry):
                r = i * 16
                vals = []
                for u in range(16):
                    for h in range(EMB // 16):
                        vals.append(
                            rows_v[b, r + u, pl.ds(h * 16, 16)] * SCALE)
                for u in range(16):
                    for h in range(EMB // 16):
                        d = (b, r // 4 + u // 4,
                             pl.ds((u % 4) * EMB + h * 16, 16))
                        r128_v[d] = vals[u * (EMB // 16) + h]
                return carry
            lax.fori_loop(0, C // 16, body, 0)

        fire(0, 0)

        def outer(c2, carry):
            for b in range(2):
                ci = c2 * 2 + b
                nb = 1 - b

                @pl.when(ci + 1 < NCH)
                def _fire_next():
                    fire(ci + 1, nb)

                wait_g(b)

                @pl.when(ci >= 2)
                def _drain_out():
                    pltpu.make_async_copy(
                        r128_v.at[b], out_slice(ci - 2), semo[b]).wait()

                repack(b)
                pltpu.async_copy(r128_v.at[b], out_slice(ci), semo[b])
            return carry

        lax.fori_loop(0, NCH // 2, outer, 0)
        pltpu.make_async_copy(r128_v.at[0], out_slice(NCH - 2), semo[0]).wait()
        pltpu.make_async_copy(r128_v.at[1], out_slice(NCH - 1), semo[1]).wait()

    return k


FC = 640
GJ = 64


def _make_formatter(B):
    info = plsc.get_sparse_core_info()
    NC = info.num_cores
    NW = NC * info.num_subcores
    RPW = B // NW
    NCHF = RPW // FC
    DR = FC * EMB // 128

    mesh = plsc.VectorSubcoreMesh(core_axis_name="c", subcore_axis_name="s")

    @pl.kernel(
        mesh=mesh,
        out_type=jax.ShapeDtypeStruct((B, EMB), jnp.float32),
        scratch_types=[
            pltpu.VMEM((2, DR, 128), jnp.float32),
            pltpu.VMEM((2, GJ, EMB), jnp.float32),
            pltpu.SemaphoreType.DMA,
            pltpu.SemaphoreType.DMA,
            pltpu.SemaphoreType.DMA,
            pltpu.SemaphoreType.DMA,
        ],
    )
    def k(in_hbm, out_hbm, vin, vout, sr0, sr1, sw0, sw1):
        wid = lax.axis_index("s") * NC + lax.axis_index("c")
        row0 = wid * RPW
        semr = (sr0, sr1)
        semw = (sw0, sw1)

        def src(c):
            r0 = pl.multiple_of((row0 + c * FC) * EMB // 128, 8)
            return in_hbm.at[pl.ds(r0, DR)]

        def repack(b, sb, g):
            def body(i, carry):
                j0 = i * 8
                vals = []
                for u in range(8):
                    for h in range(EMB // 16):
                        vals.append(vin[
                            b, g * (GJ * EMB // 128) + i * 2 + u // 4,
                            pl.ds((u % 4) * EMB + h * 16, 16)])
                for u in range(8):
                    for h in range(EMB // 16):
                        vout[sb, j0 + u, pl.ds(h * 16, 16)] = \
                            vals[u * (EMB // 16) + h]
                return carry
            lax.fori_loop(0, GJ // 8, body, 0)

        pltpu.async_copy(src(0), vin.at[0], semr[0])

        def outer(c2, carry):
            for half in range(2):
                c = c2 * 2 + half
                b = half
                nb = 1 - half

                @pl.when(c + 1 < NCHF)
                def _fire_read():
                    pltpu.async_copy(src(c + 1), vin.at[nb], semr[nb])

                pltpu.make_async_copy(src(c), vin.at[b], semr[b]).wait()

                for g in range(FC // GJ):
                    sb = g & 1
                    r0 = pl.multiple_of(row0 + c * FC + g * GJ, 8)
                    dstv = out_hbm.at[pl.ds(r0, GJ)]

                    @pl.when((c > 0) | (g >= 2))
                    def _drain_w():
                        pltpu.make_async_copy(
                            vout.at[sb], dstv, semw[sb]).wait()
                    repack(b, sb, g)
                    pltpu.async_copy(vout.at[sb], dstv, semw[sb])
            return carry

        lax.fori_loop(0, NCHF // 2, outer, 0)
        last = out_hbm.at[pl.ds(pl.multiple_of(row0, 8), GJ)]
        pltpu.make_async_copy(vout.at[0], last, semw[0]).wait()
        pltpu.make_async_copy(vout.at[1], last, semw[1]).wait()

    return k


def kernel(tokens, embedding):
    B = tokens.shape[0] * tokens.shape[1]
    V = embedding.shape[0]
    tok2d = tokens.reshape(B // K, K).astype(jnp.int32)
    out128 = _make_gather(B, V)(tok2d, embedding)
    out2d = _make_formatter(B)(out128)
    return out2d.reshape(tokens.shape[0], tokens.shape[1], EMB)


# speedup vs baseline: 1.1809x; 1.0002x over previous
import math

import jax
import jax.numpy as jnp
from jax import lax
from jax.experimental import pallas as pl
from jax.experimental.pallas import tpu as pltpu
from jax.experimental.pallas import tpu_sc as plsc

EMB = 32
SCALE = math.sqrt(EMB)


K = 128
CH = 5
C = K * CH


def _make_gather(B, V):
    info = plsc.get_sparse_core_info()
    NC = info.num_cores
    NW = NC * info.num_subcores
    RW = B // NW
    NCH = RW // C
    assert NCH % 2 == 0 and NCH * C == RW

    mesh = plsc.VectorSubcoreMesh(core_axis_name="c", subcore_axis_name="s")

    @pl.kernel(
        mesh=mesh,
        out_type=jax.ShapeDtypeStruct((B * EMB // 128, 128), jnp.float32),
        scratch_types=[
            pltpu.VMEM((RW // K, K), jnp.int32),
            pltpu.VMEM((2, C, EMB), jnp.float32),
            pltpu.VMEM((2, C * EMB // 128, 128), jnp.float32),
            pltpu.SemaphoreType.DMA,
            pltpu.SemaphoreType.DMA,
            pltpu.SemaphoreType.DMA,
            pltpu.SemaphoreType.DMA,
        ],
        compiler_params=pltpu.CompilerParams(use_tc_tiling_on_sc=False),
    )
    def k(tok_hbm, tbl_hbm, out_hbm, idx_v, rows_v, r128_v, sg0, sg1, so0, so1):
        wid = lax.axis_index("s") * NC + lax.axis_index("c")
        base = wid * RW
        semg = (sg0, sg1)
        semo = (so0, so1)

        pltpu.sync_copy(
            tok_hbm.at[pl.ds(pl.multiple_of(base // K, 8), RW // K)], idx_v)

        def fire(ci, b):
            for j in range(CH):
                pltpu.async_copy(
                    tbl_hbm.at[idx_v.at[ci * CH + j]],
                    rows_v.at[b, pl.ds(j * K, K)],
                    semg[b],
                )

        def wait_g(b):
            pltpu.make_async_copy(
                tbl_hbm.at[pl.ds(0, C)], rows_v.at[b], semg[b]).wait()

        def out_slice(ci):
            r0 = pl.multiple_of((base + ci * C) * EMB // 128, 8)
            return out_hbm.at[pl.ds(r0, C * EMB // 128)]

        def repack(b):
            def body(i, carry):
                r = i * 16
                vals = []
                for u in range(16):
                    for h in range(EMB // 16):
                        vals.append(
                            rows_v[b, r + u, pl.ds(h * 16, 16)] * SCALE)
                for u in range(16):
                    for h in range(EMB // 16):
                        d = (b, r // 4 + u // 4,
                             pl.ds((u % 4) * EMB + h * 16, 16))
                        r128_v[d] = vals[u * (EMB // 16) + h]
                return carry
            lax.fori_loop(0, C // 16, body, 0)

        fire(0, 0)

        def outer(c2, carry):
            for b in range(2):
                ci = c2 * 2 + b
                nb = 1 - b

                @pl.when(ci + 1 < NCH)
                def _fire_next():
                    fire(ci + 1, nb)

                wait_g(b)

                @pl.when(ci >= 2)
                def _drain_out():
                    pltpu.make_async_copy(
                        r128_v.at[b], out_slice(ci - 2), semo[b]).wait()

                repack(b)
                pltpu.async_copy(r128_v.at[b], out_slice(ci), semo[b])
            return carry

        lax.fori_loop(0, NCH // 2, outer, 0)
        pltpu.make_async_copy(r128_v.at[0], out_slice(NCH - 2), semo[0]).wait()
        pltpu.make_async_copy(r128_v.at[1], out_slice(NCH - 1), semo[1]).wait()

    return k


FC = 640
GJ = 160


def _make_formatter(B):
    info = plsc.get_sparse_core_info()
    NC = info.num_cores
    NW = NC * info.num_subcores
    RPW = B // NW
    NCHF = RPW // FC
    DR = FC * EMB // 128

    mesh = plsc.VectorSubcoreMesh(core_axis_name="c", subcore_axis_name="s")

    @pl.kernel(
        mesh=mesh,
        out_type=jax.ShapeDtypeStruct((B, EMB), jnp.float32),
        scratch_types=[
            pltpu.VMEM((2, DR, 128), jnp.float32),
            pltpu.VMEM((2, GJ, EMB), jnp.float32),
            pltpu.SemaphoreType.DMA,
            pltpu.SemaphoreType.DMA,
            pltpu.SemaphoreType.DMA,
            pltpu.SemaphoreType.DMA,
        ],
    )
    def k(in_hbm, out_hbm, vin, vout, sr0, sr1, sw0, sw1):
        wid = lax.axis_index("s") * NC + lax.axis_index("c")
        row0 = wid * RPW
        semr = (sr0, sr1)
        semw = (sw0, sw1)

        def src(c):
            r0 = pl.multiple_of((row0 + c * FC) * EMB // 128, 8)
            return in_hbm.at[pl.ds(r0, DR)]

        def repack(b, sb, g):
            def body(i, carry):
                j0 = i * 16
                vals = []
                for u in range(16):
                    for h in range(EMB // 16):
                        vals.append(vin[
                            b, g * (GJ * EMB // 128) + i * 4 + u // 4,
                            pl.ds((u % 4) * EMB + h * 16, 16)])
                for u in range(16):
                    for h in range(EMB // 16):
                        vout[sb, j0 + u, pl.ds(h * 16, 16)] = \
                            vals[u * (EMB // 16) + h]
                return carry
            lax.fori_loop(0, GJ // 16, body, 0)

        pltpu.async_copy(src(0), vin.at[0], semr[0])

        def outer(c2, carry):
            for half in range(2):
                c = c2 * 2 + half
                b = half
                nb = 1 - half

                @pl.when(c + 1 < NCHF)
                def _fire_read():
                    pltpu.async_copy(src(c + 1), vin.at[nb], semr[nb])

                pltpu.make_async_copy(src(c), vin.at[b], semr[b]).wait()

                for g in range(FC // GJ):
                    sb = g & 1
                    r0 = pl.multiple_of(row0 + c * FC + g * GJ, 8)
                    dstv = out_hbm.at[pl.ds(r0, GJ)]

                    @pl.when((c > 0) | (g >= 2))
                    def _drain_w():
                        pltpu.make_async_copy(
                            vout.at[sb], dstv, semw[sb]).wait()
                    repack(b, sb, g)
                    pltpu.async_copy(vout.at[sb], dstv, semw[sb])
            return carry

        lax.fori_loop(0, NCHF // 2, outer, 0)
        last = out_hbm.at[pl.ds(pl.multiple_of(row0, 8), GJ)]
        pltpu.make_async_copy(vout.at[0], last, semw[0]).wait()
        pltpu.make_async_copy(vout.at[1], last, semw[1]).wait()

    return k


def kernel(tokens, embedding):
    B = tokens.shape[0] * tokens.shape[1]
    V = embedding.shape[0]
    tok2d = tokens.reshape(B // K, K).astype(jnp.int32)
    out128 = _make_gather(B, V)(tok2d, embedding)
    out2d = _make_formatter(B)(out128)
    return out2d.reshape(tokens.shape[0], tokens.shape[1], EMB)
